# baseline (speedup 1.0000x reference)
import jax
import jax.numpy as jnp
from jax.experimental import pallas as pl
from jax.experimental.pallas import tpu as pltpu

_WEIGHT_POINT = 100.0
_WEIGHT_EDGE = 100.0


def _loss_body(p_ref, t_ref, m_ref, e_ref, y_ref, n_ref, o_ref, *, cp, ce):
    step = pl.program_id(0)

    @pl.when(step == 0)
    def _init():
        o_ref[0, 0] = 0.0

    p = p_ref[...]
    t = t_ref[...]
    m = m_ref[...]
    d = p * m - t * m
    s_point = jnp.sum(d * d, dtype=jnp.float32)

    e = e_ref[...]
    y = y_ref[...]
    n = n_ref[...]
    ii = jax.lax.broadcasted_iota(jnp.int32, e.shape, 1)
    jj = jax.lax.broadcasted_iota(jnp.int32, e.shape, 2)
    valid = (ii < n) & (jj < n)
    tgt = jj == y
    log_p = jnp.maximum(jnp.log(e), -100.0)
    log_1mp = jnp.maximum(jnp.log(1.0 - e), -100.0)
    bce = -jnp.where(tgt, log_p, log_1mp)
    s_edge = jnp.sum(jnp.where(valid, bce, 0.0))

    o_ref[0, 0] += cp * s_point + ce * s_edge


def kernel(points_pred, targets, mask, edges_pred, match_targets, npoints):
    B, C, H, W = points_pred.shape
    F = C * H * W
    M = match_targets.shape[1]

    e3 = edges_pred.reshape(B, M, M)
    n3 = npoints.reshape(B, 1, 1)

    nb = 8
    bc = B // nb

    cp = _WEIGHT_POINT / (B * F)
    ce = _WEIGHT_EDGE / (B * M * M)

    import functools
    body = functools.partial(_loss_body, cp=cp, ce=ce)

    out = pl.pallas_call(
        body,
        grid=(nb,),
        in_specs=[
            pl.BlockSpec((bc, C, H, W), lambda i: (i, 0, 0, 0)),
            pl.BlockSpec((bc, C, H, W), lambda i: (i, 0, 0, 0)),
            pl.BlockSpec((bc, C, H, W), lambda i: (i, 0, 0, 0)),
            pl.BlockSpec((bc, M, M), lambda i: (i, 0, 0)),
            pl.BlockSpec((bc, M, 1), lambda i: (i, 0, 0)),
            pl.BlockSpec((bc, 1, 1), lambda i: (i, 0, 0)),
        ],
        out_specs=pl.BlockSpec(
            (1, 1), lambda i: (0, 0), memory_space=pltpu.SMEM
        ),
        out_shape=jax.ShapeDtypeStruct((1, 1), jnp.float32),
    )(points_pred, targets, mask, e3, match_targets, n3)
    return out.reshape(())


# speedup vs baseline: 17.8618x; 17.8618x over previous
import functools

import jax
import jax.numpy as jnp
from jax.experimental import pallas as pl
from jax.experimental.pallas import tpu as pltpu

_WEIGHT_POINT = 100.0
_WEIGHT_EDGE = 100.0


def _loss_body(p_ref, t_ref, m_ref, e_ref, y_ref, n_ref, o_ref, *, cp, ce):
    step = pl.program_id(0)

    @pl.when(step == 0)
    def _init():
        o_ref[0, 0] = 0.0

    d = (p_ref[...] - t_ref[...]) * m_ref[...]
    s_point = jnp.sum(d * d, dtype=jnp.float32)
    o_ref[0, 0] += cp * s_point

    @pl.when(step == 0)
    def _edge():
        e = e_ref[...]
        y = y_ref[...]
        n = n_ref[...]
        ii = jax.lax.broadcasted_iota(jnp.int32, e.shape, 0)
        jj = jax.lax.broadcasted_iota(jnp.int32, e.shape, 1)
        nb = n[None, None]
        valid = (ii < nb) & (jj < nb)
        tgt = jj == y[:, None]
        log_p = jnp.maximum(jnp.log(e), -100.0)
        log_1mp = jnp.maximum(jnp.log(1.0 - e), -100.0)
        bce = -jnp.where(tgt, log_p, log_1mp)
        s_edge = jnp.sum(jnp.where(valid, bce, 0.0), dtype=jnp.float32)
        o_ref[0, 0] += ce * s_edge


def kernel(points_pred, targets, mask, edges_pred, match_targets, npoints):
    B, C, H, W = points_pred.shape
    F = C * H * W
    M = match_targets.shape[1]
    S, L = 8, B // 8

    pt = jnp.transpose(points_pred, (1, 2, 3, 0)).reshape(F, B)
    tt = jnp.transpose(targets, (1, 2, 3, 0)).reshape(F, B)
    mt = jnp.transpose(mask, (1, 2, 3, 0)).reshape(F, B)
    e4 = jnp.transpose(edges_pred, (2, 1, 0)).reshape(M, M, S, L)
    y3 = jnp.transpose(match_targets, (1, 2, 0)).reshape(M, S, L)
    n2 = npoints.reshape(S, L)

    nsteps = 8
    rows = F // nsteps

    cp = _WEIGHT_POINT / (B * F)
    ce = _WEIGHT_EDGE / (B * M * M)
    body = functools.partial(_loss_body, cp=cp, ce=ce)

    out = pl.pallas_call(
        body,
        grid=(nsteps,),
        in_specs=[
            pl.BlockSpec((rows, B), lambda i: (i, 0)),
            pl.BlockSpec((rows, B), lambda i: (i, 0)),
            pl.BlockSpec((rows, B), lambda i: (i, 0)),
            pl.BlockSpec((M, M, S, L), lambda i: (0, 0, 0, 0)),
            pl.BlockSpec((M, S, L), lambda i: (0, 0, 0)),
            pl.BlockSpec((S, L), lambda i: (0, 0)),
        ],
        out_specs=pl.BlockSpec(
            (1, 1), lambda i: (0, 0), memory_space=pltpu.SMEM
        ),
        out_shape=jax.ShapeDtypeStruct((1, 1), jnp.float32),
    )(pt, tt, mt, e4, y3, n2)
    return out.reshape(())
